# baseline (speedup 1.0000x reference)
import jax
import jax.numpy as jnp
from jax.experimental import pallas as pl
from jax.experimental.pallas import tpu as pltpu

_H = 4
_C = 128
_N = 512
_IN_F = 1024
_NC = 1000
_KCH = 256
_NKC = _IN_F // _KCH
_RB = 128
_NRB = _N // _RB

_DNT = (((1,), (1,)), ((), ()))


def _matmul_t(x, w):
    return jax.lax.dot_general(x, w, _DNT, preferred_element_type=jnp.float32)


def _head_ef(Wh, a_ref, hd):
    Wh_h = Wh[:, hd * _C:(hd + 1) * _C]
    e = _matmul_t(Wh_h, a_ref[hd:hd + 1, :_C])
    f = _matmul_t(a_ref[hd:hd + 1, _C:], Wh_h)
    return Wh_h, e, f


def _masked_softmax_agg(e_blk, f, mask_blk, Wh_h):
    maxf = jnp.max(f)
    eb = e_blk + maxf
    mprime = jnp.where(eb >= 0, eb, 0.01 * eb)
    logits = e_blk + f
    logits = jnp.where(logits >= 0, logits, 0.01 * logits)
    p = jnp.where(mask_blk, jnp.exp(logits - mprime), 0.0)
    s = jnp.sum(p, axis=1, keepdims=True)
    agg = jnp.dot(p, Wh_h, preferred_element_type=jnp.float32)
    return agg * (1.0 / s)


def _attention(Wh, mask, a_ref):
    outs = []
    for hd in range(_H):
        Wh_h, e, f = _head_ef(Wh, a_ref, hd)
        outs.append(_masked_softmax_agg(e, f, mask, Wh_h))
    return jnp.concatenate(outs, axis=1)


def _gat_kernel(h_hbm, A_hbm, w1_hbm, b1_hbm, a1_hbm, w2_hbm, b2_hbm,
                a2_hbm, w3_hbm, b3_hbm, fl_hbm, flb_hbm, h3_hbm, out_hbm,
                h_v, A_v, w1_v, b1_v, a1_v, w2_v, b2_v, a2_v, w3_v, b3_v,
                fl_v, flb_v, h3_v, out_v,
                s_h, s_w1, s_A, s_b1, s_a1, s_w2, s_b2, s_a2, s_w3, s_b3,
                s_fl, s_flb, s_h3o, s_outo):
    cp_h = [pltpu.make_async_copy(
        h_hbm.at[0, :, k * _KCH:(k + 1) * _KCH],
        h_v.at[:, k * _KCH:(k + 1) * _KCH], s_h.at[k]) for k in range(_NKC)]
    cp_w1 = [pltpu.make_async_copy(
        w1_hbm.at[:, k * _KCH:(k + 1) * _KCH],
        w1_v.at[:, k * _KCH:(k + 1) * _KCH], s_w1.at[k]) for k in range(_NKC)]
    cp_b1 = pltpu.make_async_copy(b1_hbm, b1_v, s_b1)
    cp_A = pltpu.make_async_copy(A_hbm.at[0], A_v, s_A)
    cp_a1 = pltpu.make_async_copy(a1_hbm, a1_v, s_a1)
    cp_w3 = pltpu.make_async_copy(w3_hbm, w3_v, s_w3)
    cp_b3 = pltpu.make_async_copy(b3_hbm, b3_v, s_b3)
    cp_w2 = pltpu.make_async_copy(w2_hbm, w2_v, s_w2)
    cp_b2 = pltpu.make_async_copy(b2_hbm, b2_v, s_b2)
    cp_a2 = pltpu.make_async_copy(a2_hbm, a2_v, s_a2)
    cp_fl = pltpu.make_async_copy(fl_hbm, fl_v, s_fl)
    cp_flb = pltpu.make_async_copy(flb_hbm, flb_v, s_flb)
    for k in range(_NKC):
        cp_h[k].start()
        cp_w1[k].start()
    for cp in (cp_A, cp_a1, cp_b1, cp_w3, cp_b3, cp_w2, cp_b2, cp_a2,
               cp_fl, cp_flb):
        cp.start()

    cp_h[0].wait()
    cp_w1[0].wait()
    sl = pl.ds(0, _KCH)
    Wh1 = _matmul_t(h_v[:, sl], w1_v[:, sl])
    for k in range(1, _NKC):
        cp_h[k].wait()
        cp_w1[k].wait()
        sl = pl.ds(k * _KCH, _KCH)
        Wh1 = Wh1 + _matmul_t(h_v[:, sl], w1_v[:, sl])
    cp_b1.wait()
    Wh1 = Wh1 + b1_v[...]

    cp_A.wait()
    mask = A_v[...] != 0
    cp_a1.wait()
    h1 = _attention(Wh1, mask, a1_v)

    cp_w3.wait()
    cp_b3.wait()
    h2 = _matmul_t(h1, w3_v[...]) + b3_v[...]
    cp_w2.wait()
    cp_b2.wait()
    Wh2 = _matmul_t(h2, w2_v[...]) + b2_v[...]

    cp_a2.wait()
    heads = [_head_ef(Wh2, a2_v, hd) for hd in range(_H)]
    cp_fl.wait()
    cp_flb.wait()
    out_cps = []
    for r in range(_NRB):
        rows = slice(r * _RB, (r + 1) * _RB)
        h3_blk = jnp.concatenate(
            [_masked_softmax_agg(e[rows, :], f, mask[rows, :], Wh_h)
             for (Wh_h, e, f) in heads], axis=1)
        h3_v[0, rows, :] = h3_blk
        cp1 = pltpu.make_async_copy(
            h3_v.at[:, r * _RB:(r + 1) * _RB, :],
            h3_hbm.at[:, r * _RB:(r + 1) * _RB, :], s_h3o.at[r])
        cp1.start()
        out_v[0, rows, :] = _matmul_t(h3_blk, fl_v[...]) + flb_v[...]
        cp2 = pltpu.make_async_copy(
            out_v.at[:, r * _RB:(r + 1) * _RB, :],
            out_hbm.at[:, r * _RB:(r + 1) * _RB, :], s_outo.at[r])
        cp2.start()
        out_cps.append(cp1)
        out_cps.append(cp2)
    for cp in out_cps:
        cp.wait()


def kernel(h, A, W1_w, W1_b, a1, W2_w, W2_b, a2, W3_w, W3_b, FL_w, FL_b):
    b, n, in_f = h.shape
    nc = FL_w.shape[0]
    f32 = jnp.float32
    vmem = pltpu.MemorySpace.VMEM
    any_spec = pl.BlockSpec(memory_space=pl.ANY)
    out_shapes = (
        jax.ShapeDtypeStruct((b, n, _H * _C), f32),
        jax.ShapeDtypeStruct((b, n, nc), f32),
    )
    return pl.pallas_call(
        _gat_kernel,
        out_shape=out_shapes,
        in_specs=[any_spec] * 12,
        out_specs=(any_spec, any_spec),
        scratch_shapes=(
            [vmem((n, in_f), f32),
             vmem((n, n), jnp.int32),
             vmem((_H * _C, in_f), f32),
             vmem((_H * _C,), f32),
             vmem((_H, 2 * _C), f32),
             vmem((_H * _C, _C), f32),
             vmem((_H * _C,), f32),
             vmem((_H, 2 * _C), f32),
             vmem((_C, _H * _C), f32),
             vmem((_C,), f32),
             vmem((nc, _H * _C), f32),
             vmem((nc,), f32),
             vmem((1, n, _H * _C), f32),
             vmem((1, n, nc), f32)]
            + [pltpu.SemaphoreType.DMA((_NKC,))] * 2
            + [pltpu.SemaphoreType.DMA] * 10
            + [pltpu.SemaphoreType.DMA((_NRB,))] * 2
        ),
    )(h, A, W1_w, W1_b, a1, W2_w, W2_b, a2, W3_w, W3_b, FL_w, FL_b)


# speedup vs baseline: 1.0392x; 1.0392x over previous
import jax
import jax.numpy as jnp
from jax.experimental import pallas as pl
from jax.experimental.pallas import tpu as pltpu

_H = 4
_C = 128
_N = 512
_IN_F = 1024
_NC = 1000
_KCH = 512
_NKC = _IN_F // _KCH
_RB = 128
_NRB = _N // _RB

_DNT = (((1,), (1,)), ((), ()))


def _matmul_t(x, w):
    return jax.lax.dot_general(x, w, _DNT, preferred_element_type=jnp.float32)


def _head_ef(Wh, a_ref, hd):
    Wh_h = Wh[:, hd * _C:(hd + 1) * _C]
    e = _matmul_t(Wh_h, a_ref[hd:hd + 1, :_C])
    f = _matmul_t(a_ref[hd:hd + 1, _C:], Wh_h)
    return Wh_h, e, f


def _masked_softmax_agg(e_blk, f, mask_blk, Wh_h):
    logits = e_blk + f
    logits = jnp.where(logits >= 0, logits, 0.01 * logits)
    att = jnp.where(mask_blk, logits, -jnp.inf)
    m = jnp.max(att, axis=1, keepdims=True)
    p = jnp.exp(att - m)
    s = jnp.sum(p, axis=1, keepdims=True)
    agg = jnp.dot(p, Wh_h, preferred_element_type=jnp.float32)
    return agg * (1.0 / s)


def _attention(Wh, mask, a_ref):
    outs = []
    for hd in range(_H):
        Wh_h, e, f = _head_ef(Wh, a_ref, hd)
        outs.append(_masked_softmax_agg(e, f, mask, Wh_h))
    return jnp.concatenate(outs, axis=1)


def _gat_kernel(h_hbm, A_hbm, w1_hbm, b1_hbm, a1_hbm, w2_hbm, b2_hbm,
                a2_hbm, w3_hbm, b3_hbm, fl_hbm, flb_hbm, h3_hbm, out_hbm,
                h_v, A_v, w1_v, b1_v, a1_v, w2_v, b2_v, a2_v, w3_v, b3_v,
                fl_v, flb_v, h3_v, out_v,
                s_h, s_w1, s_A, s_b1, s_a1, s_w2, s_b2, s_a2, s_w3, s_b3,
                s_fl, s_flb, s_h3o, s_outo):
    cp_h = [pltpu.make_async_copy(
        h_hbm.at[0, :, k * _KCH:(k + 1) * _KCH],
        h_v.at[:, k * _KCH:(k + 1) * _KCH], s_h.at[k]) for k in range(_NKC)]
    cp_w1 = [pltpu.make_async_copy(
        w1_hbm.at[:, k * _KCH:(k + 1) * _KCH],
        w1_v.at[:, k * _KCH:(k + 1) * _KCH], s_w1.at[k]) for k in range(_NKC)]
    cp_b1 = pltpu.make_async_copy(b1_hbm, b1_v, s_b1)
    cp_A = pltpu.make_async_copy(A_hbm.at[0], A_v, s_A)
    cp_a1 = pltpu.make_async_copy(a1_hbm, a1_v, s_a1)
    cp_w3 = pltpu.make_async_copy(w3_hbm, w3_v, s_w3)
    cp_b3 = pltpu.make_async_copy(b3_hbm, b3_v, s_b3)
    cp_w2 = pltpu.make_async_copy(w2_hbm, w2_v, s_w2)
    cp_b2 = pltpu.make_async_copy(b2_hbm, b2_v, s_b2)
    cp_a2 = pltpu.make_async_copy(a2_hbm, a2_v, s_a2)
    cp_fl = pltpu.make_async_copy(fl_hbm, fl_v, s_fl)
    cp_flb = pltpu.make_async_copy(flb_hbm, flb_v, s_flb)
    for k in range(_NKC):
        cp_h[k].start()
        cp_w1[k].start()
    for cp in (cp_A, cp_a1, cp_b1, cp_w3, cp_b3, cp_w2, cp_b2, cp_a2,
               cp_fl, cp_flb):
        cp.start()

    cp_h[0].wait()
    cp_w1[0].wait()
    sl = pl.ds(0, _KCH)
    Wh1 = _matmul_t(h_v[:, sl], w1_v[:, sl])
    for k in range(1, _NKC):
        cp_h[k].wait()
        cp_w1[k].wait()
        sl = pl.ds(k * _KCH, _KCH)
        Wh1 = Wh1 + _matmul_t(h_v[:, sl], w1_v[:, sl])
    cp_b1.wait()
    Wh1 = Wh1 + b1_v[...]

    cp_A.wait()
    mask = A_v[...] != 0
    cp_a1.wait()
    h1 = _attention(Wh1, mask, a1_v)

    cp_w3.wait()
    cp_b3.wait()
    h2 = _matmul_t(h1, w3_v[...]) + b3_v[...]
    cp_w2.wait()
    cp_b2.wait()
    Wh2 = _matmul_t(h2, w2_v[...]) + b2_v[...]

    cp_a2.wait()
    heads = [_head_ef(Wh2, a2_v, hd) for hd in range(_H)]
    cp_fl.wait()
    cp_flb.wait()
    out_cps = []
    for r in range(_NRB):
        rows = slice(r * _RB, (r + 1) * _RB)
        h3_blk = jnp.concatenate(
            [_masked_softmax_agg(e[rows, :], f, mask[rows, :], Wh_h)
             for (Wh_h, e, f) in heads], axis=1)
        h3_v[0, rows, :] = h3_blk
        cp1 = pltpu.make_async_copy(
            h3_v.at[:, r * _RB:(r + 1) * _RB, :],
            h3_hbm.at[:, r * _RB:(r + 1) * _RB, :], s_h3o.at[r])
        cp1.start()
        out_v[0, rows, :] = _matmul_t(h3_blk, fl_v[...]) + flb_v[...]
        cp2 = pltpu.make_async_copy(
            out_v.at[:, r * _RB:(r + 1) * _RB, :],
            out_hbm.at[:, r * _RB:(r + 1) * _RB, :], s_outo.at[r])
        cp2.start()
        out_cps.append(cp1)
        out_cps.append(cp2)
    for cp in out_cps:
        cp.wait()


def kernel(h, A, W1_w, W1_b, a1, W2_w, W2_b, a2, W3_w, W3_b, FL_w, FL_b):
    b, n, in_f = h.shape
    nc = FL_w.shape[0]
    f32 = jnp.float32
    vmem = pltpu.MemorySpace.VMEM
    any_spec = pl.BlockSpec(memory_space=pl.ANY)
    out_shapes = (
        jax.ShapeDtypeStruct((b, n, _H * _C), f32),
        jax.ShapeDtypeStruct((b, n, nc), f32),
    )
    return pl.pallas_call(
        _gat_kernel,
        out_shape=out_shapes,
        in_specs=[any_spec] * 12,
        out_specs=(any_spec, any_spec),
        scratch_shapes=(
            [vmem((n, in_f), f32),
             vmem((n, n), jnp.int32),
             vmem((_H * _C, in_f), f32),
             vmem((_H * _C,), f32),
             vmem((_H, 2 * _C), f32),
             vmem((_H * _C, _C), f32),
             vmem((_H * _C,), f32),
             vmem((_H, 2 * _C), f32),
             vmem((_C, _H * _C), f32),
             vmem((_C,), f32),
             vmem((nc, _H * _C), f32),
             vmem((nc,), f32),
             vmem((1, n, _H * _C), f32),
             vmem((1, n, nc), f32)]
            + [pltpu.SemaphoreType.DMA((_NKC,))] * 2
            + [pltpu.SemaphoreType.DMA] * 10
            + [pltpu.SemaphoreType.DMA((_NRB,))] * 2
        ),
    )(h, A, W1_w, W1_b, a1, W2_w, W2_b, a2, W3_w, W3_b, FL_w, FL_b)


# speedup vs baseline: 1.0597x; 1.0197x over previous
import jax
import jax.numpy as jnp
from jax.experimental import pallas as pl
from jax.experimental.pallas import tpu as pltpu

_H = 4
_C = 128
_N = 512
_IN_F = 1024
_NC = 1000
_KCH = 256
_NKC = _IN_F // _KCH
_RB = 128
_NRB = _N // _RB

_DNT = (((1,), (1,)), ((), ()))


def _matmul_t(x, w):
    return jax.lax.dot_general(x, w, _DNT, preferred_element_type=jnp.float32)


def _head_ef(Wh, a_ref, hd):
    Wh_h = Wh[:, hd * _C:(hd + 1) * _C]
    e = _matmul_t(Wh_h, a_ref[hd:hd + 1, :_C])
    f = _matmul_t(a_ref[hd:hd + 1, _C:], Wh_h)
    return Wh_h, e, f


def _masked_softmax_agg(e_blk, f, mask_blk, Wh_h):
    logits = e_blk + f
    logits = jnp.where(logits >= 0, logits, 0.01 * logits)
    att = jnp.where(mask_blk, logits, -jnp.inf)
    m = jnp.max(att, axis=1, keepdims=True)
    p = jnp.exp(att - m)
    s = jnp.sum(p, axis=1, keepdims=True)
    agg = jnp.dot(p, Wh_h, preferred_element_type=jnp.float32,
                  precision=jax.lax.Precision.DEFAULT)
    return agg * (1.0 / s)


def _attention(Wh, mask, a_ref):
    outs = []
    for hd in range(_H):
        Wh_h, e, f = _head_ef(Wh, a_ref, hd)
        outs.append(_masked_softmax_agg(e, f, mask, Wh_h))
    return jnp.concatenate(outs, axis=1)


def _gat_kernel(h_hbm, A_hbm, w1_hbm, b1_hbm, a1_hbm, w2_hbm, b2_hbm,
                a2_hbm, w3_hbm, b3_hbm, fl_hbm, flb_hbm, h3_hbm, out_hbm,
                h_v, A_v, w1_v, b1_v, a1_v, w2_v, b2_v, a2_v, w3_v, b3_v,
                fl_v, flb_v, h3_v, out_v,
                s_h, s_w1, s_A, s_b1, s_a1, s_w2, s_b2, s_a2, s_w3, s_b3,
                s_fl, s_flb, s_h3o, s_outo):
    cp_h = [pltpu.make_async_copy(
        h_hbm.at[0, :, k * _KCH:(k + 1) * _KCH],
        h_v.at[:, k * _KCH:(k + 1) * _KCH], s_h.at[k]) for k in range(_NKC)]
    cp_w1 = [pltpu.make_async_copy(
        w1_hbm.at[:, k * _KCH:(k + 1) * _KCH],
        w1_v.at[:, k * _KCH:(k + 1) * _KCH], s_w1.at[k]) for k in range(_NKC)]
    cp_b1 = pltpu.make_async_copy(b1_hbm, b1_v, s_b1)
    cp_A = pltpu.make_async_copy(A_hbm.at[0], A_v, s_A)
    cp_a1 = pltpu.make_async_copy(a1_hbm, a1_v, s_a1)
    cp_w3 = pltpu.make_async_copy(w3_hbm, w3_v, s_w3)
    cp_b3 = pltpu.make_async_copy(b3_hbm, b3_v, s_b3)
    cp_w2 = pltpu.make_async_copy(w2_hbm, w2_v, s_w2)
    cp_b2 = pltpu.make_async_copy(b2_hbm, b2_v, s_b2)
    cp_a2 = pltpu.make_async_copy(a2_hbm, a2_v, s_a2)
    cp_fl = pltpu.make_async_copy(fl_hbm, fl_v, s_fl)
    cp_flb = pltpu.make_async_copy(flb_hbm, flb_v, s_flb)
    for k in range(_NKC):
        cp_h[k].start()
        cp_w1[k].start()
    for cp in (cp_A, cp_a1, cp_b1, cp_w3, cp_b3, cp_w2, cp_b2, cp_a2,
               cp_fl, cp_flb):
        cp.start()

    cp_h[0].wait()
    cp_w1[0].wait()
    sl = pl.ds(0, _KCH)
    Wh1 = _matmul_t(h_v[:, sl], w1_v[:, sl])
    for k in range(1, _NKC):
        cp_h[k].wait()
        cp_w1[k].wait()
        sl = pl.ds(k * _KCH, _KCH)
        Wh1 = Wh1 + _matmul_t(h_v[:, sl], w1_v[:, sl])
    cp_b1.wait()
    Wh1 = Wh1 + b1_v[...]

    cp_A.wait()
    mask = A_v[...] != 0
    cp_a1.wait()
    h1 = _attention(Wh1, mask, a1_v)

    cp_w3.wait()
    cp_b3.wait()
    h2 = _matmul_t(h1, w3_v[...]) + b3_v[...]
    cp_w2.wait()
    cp_b2.wait()
    Wh2 = _matmul_t(h2, w2_v[...]) + b2_v[...]

    cp_a2.wait()
    heads = [_head_ef(Wh2, a2_v, hd) for hd in range(_H)]
    cp_fl.wait()
    cp_flb.wait()
    out_cps = []
    for r in range(_NRB):
        rows = slice(r * _RB, (r + 1) * _RB)
        h3_blk = jnp.concatenate(
            [_masked_softmax_agg(e[rows, :], f, mask[rows, :], Wh_h)
             for (Wh_h, e, f) in heads], axis=1)
        h3_v[0, rows, :] = h3_blk
        cp1 = pltpu.make_async_copy(
            h3_v.at[:, r * _RB:(r + 1) * _RB, :],
            h3_hbm.at[:, r * _RB:(r + 1) * _RB, :], s_h3o.at[r])
        cp1.start()
        out_v[0, rows, :] = jax.lax.dot_general(
            h3_blk, fl_v[...], _DNT, preferred_element_type=jnp.float32,
            precision=jax.lax.Precision.DEFAULT) + flb_v[...]
        cp2 = pltpu.make_async_copy(
            out_v.at[:, r * _RB:(r + 1) * _RB, :],
            out_hbm.at[:, r * _RB:(r + 1) * _RB, :], s_outo.at[r])
        cp2.start()
        out_cps.append(cp1)
        out_cps.append(cp2)
    for cp in out_cps:
        cp.wait()


def kernel(h, A, W1_w, W1_b, a1, W2_w, W2_b, a2, W3_w, W3_b, FL_w, FL_b):
    b, n, in_f = h.shape
    nc = FL_w.shape[0]
    f32 = jnp.float32
    vmem = pltpu.MemorySpace.VMEM
    any_spec = pl.BlockSpec(memory_space=pl.ANY)
    out_shapes = (
        jax.ShapeDtypeStruct((b, n, _H * _C), f32),
        jax.ShapeDtypeStruct((b, n, nc), f32),
    )
    return pl.pallas_call(
        _gat_kernel,
        out_shape=out_shapes,
        in_specs=[any_spec] * 12,
        out_specs=(any_spec, any_spec),
        scratch_shapes=(
            [vmem((n, in_f), f32),
             vmem((n, n), jnp.int32),
             vmem((_H * _C, in_f), f32),
             vmem((_H * _C,), f32),
             vmem((_H, 2 * _C), f32),
             vmem((_H * _C, _C), f32),
             vmem((_H * _C,), f32),
             vmem((_H, 2 * _C), f32),
             vmem((_C, _H * _C), f32),
             vmem((_C,), f32),
             vmem((nc, _H * _C), f32),
             vmem((nc,), f32),
             vmem((1, n, _H * _C), f32),
             vmem((1, n, nc), f32)]
            + [pltpu.SemaphoreType.DMA((_NKC,))] * 2
            + [pltpu.SemaphoreType.DMA] * 10
            + [pltpu.SemaphoreType.DMA((_NRB,))] * 2
        ),
    )(h, A, W1_w, W1_b, a1, W2_w, W2_b, a2, W3_w, W3_b, FL_w, FL_b)


# speedup vs baseline: 1.0700x; 1.0097x over previous
import jax
import jax.numpy as jnp
from jax.experimental import pallas as pl
from jax.experimental.pallas import tpu as pltpu

_H = 4
_C = 128
_N = 512
_IN_F = 1024
_NC = 1000
_KCH = 256
_NKC = _IN_F // _KCH
_RB = 128
_NRB = _N // _RB

_DNT = (((1,), (1,)), ((), ()))


def _matmul_t(x, w):
    return jax.lax.dot_general(x, w, _DNT, preferred_element_type=jnp.float32)


def _head_ef(Wh, a_ref, hd):
    Wh_h = Wh[:, hd * _C:(hd + 1) * _C]
    e = _matmul_t(Wh_h, a_ref[hd:hd + 1, :_C])
    f = _matmul_t(a_ref[hd:hd + 1, _C:], Wh_h)
    return Wh_h, e, f


def _masked_softmax_agg(e_blk, f, mask_blk, Wh_h):
    log2e = jnp.float32(1.4426950408889634)
    e2 = e_blk * log2e
    f2 = f * log2e
    logits = e2 + f2
    logits = jnp.where(logits >= 0, logits, 0.01 * logits)
    att = jnp.where(mask_blk, logits, -jnp.inf)
    m = jnp.max(att, axis=1, keepdims=True)
    p = jnp.exp2(att - m)
    s = jnp.sum(p, axis=1, keepdims=True)
    agg = jnp.dot(p, Wh_h, preferred_element_type=jnp.float32,
                  precision=jax.lax.Precision.DEFAULT)
    return agg * (1.0 / s)


def _attention(Wh, mask, a_ref):
    outs = []
    for hd in range(_H):
        Wh_h, e, f = _head_ef(Wh, a_ref, hd)
        outs.append(_masked_softmax_agg(e, f, mask, Wh_h))
    return jnp.concatenate(outs, axis=1)


def _gat_kernel(h_hbm, A_hbm, w1_hbm, b1_hbm, a1_hbm, w2_hbm, b2_hbm,
                a2_hbm, w3_hbm, b3_hbm, fl_hbm, flb_hbm, h3_hbm, out_hbm,
                h_v, A_v, w1_v, b1_v, a1_v, w2_v, b2_v, a2_v, w3_v, b3_v,
                fl_v, flb_v, h3_v, out_v,
                s_h, s_w1, s_A, s_b1, s_a1, s_w2, s_b2, s_a2, s_w3, s_b3,
                s_fl, s_flb, s_h3o, s_outo):
    cp_h = [pltpu.make_async_copy(
        h_hbm.at[0, :, k * _KCH:(k + 1) * _KCH],
        h_v.at[:, k * _KCH:(k + 1) * _KCH], s_h.at[k]) for k in range(_NKC)]
    cp_w1 = [pltpu.make_async_copy(
        w1_hbm.at[:, k * _KCH:(k + 1) * _KCH],
        w1_v.at[:, k * _KCH:(k + 1) * _KCH], s_w1.at[k]) for k in range(_NKC)]
    cp_b1 = pltpu.make_async_copy(b1_hbm, b1_v, s_b1)
    cp_A = pltpu.make_async_copy(A_hbm.at[0], A_v, s_A)
    cp_a1 = pltpu.make_async_copy(a1_hbm, a1_v, s_a1)
    cp_w3 = pltpu.make_async_copy(w3_hbm, w3_v, s_w3)
    cp_b3 = pltpu.make_async_copy(b3_hbm, b3_v, s_b3)
    cp_w2 = pltpu.make_async_copy(w2_hbm, w2_v, s_w2)
    cp_b2 = pltpu.make_async_copy(b2_hbm, b2_v, s_b2)
    cp_a2 = pltpu.make_async_copy(a2_hbm, a2_v, s_a2)
    cp_fl = pltpu.make_async_copy(fl_hbm, fl_v, s_fl)
    cp_flb = pltpu.make_async_copy(flb_hbm, flb_v, s_flb)
    for k in range(_NKC):
        cp_h[k].start()
        cp_w1[k].start()
    for cp in (cp_A, cp_a1, cp_b1, cp_w3, cp_b3, cp_w2, cp_b2, cp_a2,
               cp_fl, cp_flb):
        cp.start()

    cp_h[0].wait()
    cp_w1[0].wait()
    sl = pl.ds(0, _KCH)
    Wh1 = _matmul_t(h_v[:, sl], w1_v[:, sl])
    for k in range(1, _NKC):
        cp_h[k].wait()
        cp_w1[k].wait()
        sl = pl.ds(k * _KCH, _KCH)
        Wh1 = Wh1 + _matmul_t(h_v[:, sl], w1_v[:, sl])
    cp_b1.wait()
    Wh1 = Wh1 + b1_v[...]

    cp_A.wait()
    mask = A_v[...] != 0
    cp_a1.wait()
    h1 = _attention(Wh1, mask, a1_v)

    cp_w3.wait()
    cp_b3.wait()
    h2 = _matmul_t(h1, w3_v[...]) + b3_v[...]
    cp_w2.wait()
    cp_b2.wait()
    Wh2 = _matmul_t(h2, w2_v[...]) + b2_v[...]

    cp_a2.wait()
    heads = [_head_ef(Wh2, a2_v, hd) for hd in range(_H)]
    cp_fl.wait()
    cp_flb.wait()
    out_cps = []
    for r in range(_NRB):
        rows = slice(r * _RB, (r + 1) * _RB)
        h3_blk = jnp.concatenate(
            [_masked_softmax_agg(e[rows, :], f, mask[rows, :], Wh_h)
             for (Wh_h, e, f) in heads], axis=1)
        h3_v[0, rows, :] = h3_blk
        cp1 = pltpu.make_async_copy(
            h3_v.at[:, r * _RB:(r + 1) * _RB, :],
            h3_hbm.at[:, r * _RB:(r + 1) * _RB, :], s_h3o.at[r])
        cp1.start()
        out_v[0, rows, :] = jax.lax.dot_general(
            h3_blk, fl_v[...], _DNT, preferred_element_type=jnp.float32,
            precision=jax.lax.Precision.DEFAULT) + flb_v[...]
        cp2 = pltpu.make_async_copy(
            out_v.at[:, r * _RB:(r + 1) * _RB, :],
            out_hbm.at[:, r * _RB:(r + 1) * _RB, :], s_outo.at[r])
        cp2.start()
        out_cps.append(cp1)
        out_cps.append(cp2)
    for cp in out_cps:
        cp.wait()


def kernel(h, A, W1_w, W1_b, a1, W2_w, W2_b, a2, W3_w, W3_b, FL_w, FL_b):
    b, n, in_f = h.shape
    nc = FL_w.shape[0]
    f32 = jnp.float32
    vmem = pltpu.MemorySpace.VMEM
    any_spec = pl.BlockSpec(memory_space=pl.ANY)
    out_shapes = (
        jax.ShapeDtypeStruct((b, n, _H * _C), f32),
        jax.ShapeDtypeStruct((b, n, nc), f32),
    )
    return pl.pallas_call(
        _gat_kernel,
        out_shape=out_shapes,
        in_specs=[any_spec] * 12,
        out_specs=(any_spec, any_spec),
        scratch_shapes=(
            [vmem((n, in_f), f32),
             vmem((n, n), jnp.int32),
             vmem((_H * _C, in_f), f32),
             vmem((_H * _C,), f32),
             vmem((_H, 2 * _C), f32),
             vmem((_H * _C, _C), f32),
             vmem((_H * _C,), f32),
             vmem((_H, 2 * _C), f32),
             vmem((_C, _H * _C), f32),
             vmem((_C,), f32),
             vmem((nc, _H * _C), f32),
             vmem((nc,), f32),
             vmem((1, n, _H * _C), f32),
             vmem((1, n, nc), f32)]
            + [pltpu.SemaphoreType.DMA((_NKC,))] * 2
            + [pltpu.SemaphoreType.DMA] * 10
            + [pltpu.SemaphoreType.DMA((_NRB,))] * 2
        ),
    )(h, A, W1_w, W1_b, a1, W2_w, W2_b, a2, W3_w, W3_b, FL_w, FL_b)


# speedup vs baseline: 1.0929x; 1.0213x over previous
import jax
import jax.numpy as jnp
from jax.experimental import pallas as pl
from jax.experimental.pallas import tpu as pltpu

_H = 4
_C = 128
_N = 512
_IN_F = 1024
_NC = 1000
_KCH = 256
_NKC = _IN_F // _KCH
_RB = 128
_NRB = _N // _RB

_DNT = (((1,), (1,)), ((), ()))


def _matmul_t(x, w):
    return jax.lax.dot_general(x, w, _DNT, preferred_element_type=jnp.float32)


def _head_ef(Wh, a_ref, hd):
    Wh_h = Wh[:, hd * _C:(hd + 1) * _C]
    e = _matmul_t(Wh_h, a_ref[hd:hd + 1, :_C])
    f = _matmul_t(a_ref[hd:hd + 1, _C:], Wh_h)
    return Wh_h, e, f


def _masked_softmax_agg(e_blk, f, mask_blk, Wh_h):
    log2e = jnp.float32(1.4426950408889634)
    e2 = e_blk * log2e
    f2 = f * log2e
    logits = e2 + f2
    logits = jnp.maximum(logits, 0.01 * logits)
    att = jnp.where(mask_blk, logits, -jnp.inf)
    m = jnp.max(att, axis=1, keepdims=True)
    p = jnp.exp2(att - m)
    s = jnp.sum(p, axis=1, keepdims=True)
    agg = jnp.dot(p, Wh_h, preferred_element_type=jnp.float32,
                  precision=jax.lax.Precision.DEFAULT)
    return agg * (1.0 / s)


def _attention(Wh, mask, a_ref):
    outs = []
    for hd in range(_H):
        Wh_h, e, f = _head_ef(Wh, a_ref, hd)
        outs.append(_masked_softmax_agg(e, f, mask, Wh_h))
    return jnp.concatenate(outs, axis=1)


def _gat_kernel(h_hbm, A_hbm, w1_hbm, b1_hbm, a1_hbm, w2_hbm, b2_hbm,
                a2_hbm, w3_hbm, b3_hbm, fl_hbm, flb_hbm, h3_hbm, out_hbm,
                h_v, A_v, w1_v, b1_v, a1_v, w2_v, b2_v, a2_v, w3_v, b3_v,
                fl_v, flb_v, h3_v, out_v,
                s_h, s_w1, s_A, s_b1, s_a1, s_w2, s_b2, s_a2, s_w3, s_b3,
                s_fl, s_flb, s_h3o, s_outo):
    cp_h = [pltpu.make_async_copy(
        h_hbm.at[0, :, k * _KCH:(k + 1) * _KCH],
        h_v.at[:, k * _KCH:(k + 1) * _KCH], s_h.at[k]) for k in range(_NKC)]
    cp_w1 = [pltpu.make_async_copy(
        w1_hbm.at[:, k * _KCH:(k + 1) * _KCH],
        w1_v.at[:, k * _KCH:(k + 1) * _KCH], s_w1.at[k]) for k in range(_NKC)]
    cp_b1 = pltpu.make_async_copy(b1_hbm, b1_v, s_b1)
    cp_A = pltpu.make_async_copy(A_hbm.at[0], A_v, s_A)
    cp_a1 = pltpu.make_async_copy(a1_hbm, a1_v, s_a1)
    cp_w3 = pltpu.make_async_copy(w3_hbm, w3_v, s_w3)
    cp_b3 = pltpu.make_async_copy(b3_hbm, b3_v, s_b3)
    cp_w2 = pltpu.make_async_copy(w2_hbm, w2_v, s_w2)
    cp_b2 = pltpu.make_async_copy(b2_hbm, b2_v, s_b2)
    cp_a2 = pltpu.make_async_copy(a2_hbm, a2_v, s_a2)
    cp_fl = pltpu.make_async_copy(fl_hbm, fl_v, s_fl)
    cp_flb = pltpu.make_async_copy(flb_hbm, flb_v, s_flb)
    for k in range(_NKC):
        cp_h[k].start()
        cp_w1[k].start()
    for cp in (cp_A, cp_a1, cp_b1, cp_w3, cp_b3, cp_w2, cp_b2, cp_a2,
               cp_fl, cp_flb):
        cp.start()

    cp_h[0].wait()
    cp_w1[0].wait()
    sl = pl.ds(0, _KCH)
    Wh1 = _matmul_t(h_v[:, sl], w1_v[:, sl])
    for k in range(1, _NKC):
        cp_h[k].wait()
        cp_w1[k].wait()
        sl = pl.ds(k * _KCH, _KCH)
        Wh1 = Wh1 + _matmul_t(h_v[:, sl], w1_v[:, sl])
    cp_b1.wait()
    Wh1 = Wh1 + b1_v[...]

    cp_A.wait()
    mask = A_v[...] != 0
    cp_a1.wait()
    h1 = _attention(Wh1, mask, a1_v)

    cp_w3.wait()
    cp_b3.wait()
    h2 = _matmul_t(h1, w3_v[...]) + b3_v[...]
    cp_w2.wait()
    cp_b2.wait()
    Wh2 = _matmul_t(h2, w2_v[...]) + b2_v[...]

    cp_a2.wait()
    heads = [_head_ef(Wh2, a2_v, hd) for hd in range(_H)]
    cp_fl.wait()
    cp_flb.wait()
    out_cps = []
    for r in range(_NRB):
        rows = slice(r * _RB, (r + 1) * _RB)
        h3_blk = jnp.concatenate(
            [_masked_softmax_agg(e[rows, :], f, mask[rows, :], Wh_h)
             for (Wh_h, e, f) in heads], axis=1)
        h3_v[0, rows, :] = h3_blk
        cp1 = pltpu.make_async_copy(
            h3_v.at[:, r * _RB:(r + 1) * _RB, :],
            h3_hbm.at[:, r * _RB:(r + 1) * _RB, :], s_h3o.at[r])
        cp1.start()
        out_v[0, rows, :] = jax.lax.dot_general(
            h3_blk, fl_v[...], _DNT, preferred_element_type=jnp.float32,
            precision=jax.lax.Precision.DEFAULT) + flb_v[...]
        cp2 = pltpu.make_async_copy(
            out_v.at[:, r * _RB:(r + 1) * _RB, :],
            out_hbm.at[:, r * _RB:(r + 1) * _RB, :], s_outo.at[r])
        cp2.start()
        out_cps.append(cp1)
        out_cps.append(cp2)
    for cp in out_cps:
        cp.wait()


def kernel(h, A, W1_w, W1_b, a1, W2_w, W2_b, a2, W3_w, W3_b, FL_w, FL_b):
    b, n, in_f = h.shape
    nc = FL_w.shape[0]
    f32 = jnp.float32
    vmem = pltpu.MemorySpace.VMEM
    any_spec = pl.BlockSpec(memory_space=pl.ANY)
    out_shapes = (
        jax.ShapeDtypeStruct((b, n, _H * _C), f32),
        jax.ShapeDtypeStruct((b, n, nc), f32),
    )
    return pl.pallas_call(
        _gat_kernel,
        out_shape=out_shapes,
        in_specs=[any_spec] * 12,
        out_specs=(any_spec, any_spec),
        scratch_shapes=(
            [vmem((n, in_f), f32),
             vmem((n, n), jnp.int32),
             vmem((_H * _C, in_f), f32),
             vmem((_H * _C,), f32),
             vmem((_H, 2 * _C), f32),
             vmem((_H * _C, _C), f32),
             vmem((_H * _C,), f32),
             vmem((_H, 2 * _C), f32),
             vmem((_C, _H * _C), f32),
             vmem((_C,), f32),
             vmem((nc, _H * _C), f32),
             vmem((nc,), f32),
             vmem((1, n, _H * _C), f32),
             vmem((1, n, nc), f32)]
            + [pltpu.SemaphoreType.DMA((_NKC,))] * 2
            + [pltpu.SemaphoreType.DMA] * 10
            + [pltpu.SemaphoreType.DMA((_NRB,))] * 2
        ),
    )(h, A, W1_w, W1_b, a1, W2_w, W2_b, a2, W3_w, W3_b, FL_w, FL_b)
